# Initial kernel scaffold; baseline (speedup 1.0000x reference)
import jax
import jax.numpy as jnp
from jax.experimental import pallas as pl


def kernel(abundances, Y, M, W1, b1, W2, b2, W3, b3):
    raise NotImplementedError("write your pallas kernel here")


# speedup vs baseline: 2.2751x; 2.2751x over previous
import functools

import jax
import jax.numpy as jnp
from jax.experimental import pallas as pl
from jax.experimental.pallas import tpu as pltpu

_NEG = -1e30


def _stats_kernel(y_ref, a_ref, w1_ref, b1_ref, w2_ref, b2_ref, w3_ref,
                  mt_ref, mc_ref, m_s, s_s, v_s, c_s, *, nb, pcls):
    j = pl.program_id(0)

    @pl.when(j == 0)
    def _init():
        m_s[...] = jnp.full(m_s.shape, _NEG, jnp.float32)
        s_s[...] = jnp.zeros(s_s.shape, jnp.float32)
        v_s[...] = jnp.zeros(v_s.shape, jnp.float32)
        c_s[...] = jnp.zeros(c_s.shape, jnp.float32)

    y = y_ref[...]
    a = a_ref[...]

    h = jnp.maximum(
        jnp.dot(w1_ref[...], y, preferred_element_type=jnp.float32)
        + b1_ref[...], 0.0)
    h = jnp.maximum(
        jnp.dot(w2_ref[...], h, preferred_element_type=jnp.float32)
        + b2_ref[...], 0.0)
    logits = jnp.dot(w3_ref[...], h, preferred_element_type=jnp.float32)

    amax = jnp.max(a, axis=0, keepdims=True)
    iota = jax.lax.broadcasted_iota(jnp.int32, a.shape, 0)
    idx = jnp.min(jnp.where(a == amax, iota, pcls), axis=0, keepdims=True)
    onehot = iota == idx

    lmask = jnp.where(onehot, logits, _NEG)
    bm = jnp.max(lmask, axis=1, keepdims=True)
    m_old = m_s[:, 0:1]
    m_new = jnp.maximum(m_old, bm)
    alpha = jnp.exp(m_old - m_new)
    p = jnp.exp(jnp.where(onehot, logits - m_new, _NEG))
    s_new = s_s[:, 0:1] * alpha + jnp.sum(p, axis=1, keepdims=True)
    v_new = v_s[...] * alpha + jax.lax.dot_general(
        p, y, (((1,), (1,)), ((), ())), preferred_element_type=jnp.float32)
    c_new = c_s[:, 0:1] + jnp.sum(onehot.astype(jnp.float32), axis=1,
                                  keepdims=True)

    m_s[...] = jnp.broadcast_to(m_new, m_s.shape)
    s_s[...] = jnp.broadcast_to(s_new, s_s.shape)
    v_s[...] = v_new
    c_s[...] = jnp.broadcast_to(c_new, c_s.shape)

    @pl.when(j == nb - 1)
    def _fin():
        w = v_s[...] / s_s[:, 0:1]
        mb = jnp.maximum(mt_ref[...], 0.0)
        col = jnp.where(c_s[:, 0:1] > 10.0, 0.5 * w + 0.5 * mb, mb)
        mc_ref[...] = jnp.clip(col, 0.0, 2.0)


def _recon_kernel(mc_ref, a_ref, out_ref):
    out_ref[...] = jax.lax.dot_general(
        mc_ref[...], a_ref[...], (((0,), (0,)), ((), ())),
        preferred_element_type=jnp.float32)


@jax.jit
def kernel(abundances, Y, M, W1, b1, W2, b2, W3, b3):
    B, P, H, W_ = abundances.shape
    C = Y.shape[1]
    N = B * H * W_
    A2 = abundances.reshape(P, N)
    Y2 = Y.reshape(C, N)
    D1 = W1.shape[0]

    nblk = 4096
    nb = N // nblk

    mc_t = pl.pallas_call(
        functools.partial(_stats_kernel, nb=nb, pcls=P),
        grid=(nb,),
        in_specs=[
            pl.BlockSpec((C, nblk), lambda j: (0, j)),
            pl.BlockSpec((P, nblk), lambda j: (0, j)),
            pl.BlockSpec((D1, C), lambda j: (0, 0)),
            pl.BlockSpec((D1, 1), lambda j: (0, 0)),
            pl.BlockSpec((D1, D1), lambda j: (0, 0)),
            pl.BlockSpec((D1, 1), lambda j: (0, 0)),
            pl.BlockSpec((1, D1), lambda j: (0, 0)),
            pl.BlockSpec((P, C), lambda j: (0, 0)),
        ],
        out_specs=pl.BlockSpec((P, C), lambda j: (0, 0)),
        out_shape=jax.ShapeDtypeStruct((P, C), jnp.float32),
        scratch_shapes=[
            pltpu.VMEM((P, 128), jnp.float32),
            pltpu.VMEM((P, 128), jnp.float32),
            pltpu.VMEM((P, C), jnp.float32),
            pltpu.VMEM((P, 128), jnp.float32),
        ],
    )(Y2, A2, W1, b1.reshape(D1, 1), W2, b2.reshape(D1, 1),
      W3, M.T)

    yhat2 = pl.pallas_call(
        _recon_kernel,
        grid=(nb,),
        in_specs=[
            pl.BlockSpec((P, C), lambda j: (0, 0)),
            pl.BlockSpec((P, nblk), lambda j: (0, j)),
        ],
        out_specs=pl.BlockSpec((C, nblk), lambda j: (0, j)),
        out_shape=jax.ShapeDtypeStruct((C, N), jnp.float32),
    )(mc_t, A2)

    return yhat2.reshape(B, C, H, W_), mc_t.T


# speedup vs baseline: 2.4378x; 1.0715x over previous
import functools

import jax
import jax.numpy as jnp
from jax.experimental import pallas as pl
from jax.experimental.pallas import tpu as pltpu

_NEG = -1e30


def _stats_kernel(y_ref, a_ref, w1_ref, b1_ref, w2_ref, b2_ref, w3_ref,
                  mt_ref, mc_ref, m_s, s_s, v_s, c_s, *, nb, pcls):
    j = pl.program_id(0)

    @pl.when(j == 0)
    def _init():
        m_s[...] = jnp.full(m_s.shape, _NEG, jnp.float32)
        s_s[...] = jnp.zeros(s_s.shape, jnp.float32)
        v_s[...] = jnp.zeros(v_s.shape, jnp.float32)
        c_s[...] = jnp.zeros(c_s.shape, jnp.float32)

    y = y_ref[...]
    a = a_ref[...]

    h = jnp.maximum(
        jnp.dot(w1_ref[...], y, preferred_element_type=jnp.float32)
        + b1_ref[...], 0.0)
    h = jnp.maximum(
        jnp.dot(w2_ref[...], h, preferred_element_type=jnp.float32)
        + b2_ref[...], 0.0)
    logits = jnp.dot(w3_ref[...], h, preferred_element_type=jnp.float32)

    amax = jnp.max(a, axis=0, keepdims=True)
    iota = jax.lax.broadcasted_iota(jnp.int32, a.shape, 0)
    idx = jnp.min(jnp.where(a == amax, iota, pcls), axis=0, keepdims=True)
    onehot = iota == idx

    lmask = jnp.where(onehot, logits, _NEG)
    bm = jnp.max(lmask, axis=1, keepdims=True)
    m_old = m_s[:, 0:1]
    m_new = jnp.maximum(m_old, bm)
    alpha = jnp.exp(m_old - m_new)
    p = jnp.exp(jnp.where(onehot, logits - m_new, _NEG))
    s_new = s_s[:, 0:1] * alpha + jnp.sum(p, axis=1, keepdims=True)
    v_new = v_s[...] * alpha + jax.lax.dot_general(
        p, y, (((1,), (1,)), ((), ())), preferred_element_type=jnp.float32)
    c_new = c_s[:, 0:1] + jnp.sum(onehot.astype(jnp.float32), axis=1,
                                  keepdims=True)

    m_s[...] = jnp.broadcast_to(m_new, m_s.shape)
    s_s[...] = jnp.broadcast_to(s_new, s_s.shape)
    v_s[...] = v_new
    c_s[...] = jnp.broadcast_to(c_new, c_s.shape)

    @pl.when(j == nb - 1)
    def _fin():
        w = v_s[...] / s_s[:, 0:1]
        mb = jnp.maximum(mt_ref[...], 0.0)
        col = jnp.where(c_s[:, 0:1] > 10.0, 0.5 * w + 0.5 * mb, mb)
        mc_ref[...] = jnp.clip(col, 0.0, 2.0)


def _recon_kernel(mc_ref, a_ref, out_ref):
    out_ref[...] = jax.lax.dot_general(
        mc_ref[...], a_ref[...], (((0,), (0,)), ((), ())),
        preferred_element_type=jnp.float32)


@jax.jit
def kernel(abundances, Y, M, W1, b1, W2, b2, W3, b3):
    B, P, H, W_ = abundances.shape
    C = Y.shape[1]
    N = B * H * W_
    A2 = abundances.reshape(P, N)
    Y2 = Y.reshape(C, N)
    D1 = W1.shape[0]

    nblk = 8192
    nb = N // nblk
    nblk2 = 16384
    nb2 = N // nblk2

    mc_t = pl.pallas_call(
        functools.partial(_stats_kernel, nb=nb, pcls=P),
        grid=(nb,),
        in_specs=[
            pl.BlockSpec((C, nblk), lambda j: (0, j)),
            pl.BlockSpec((P, nblk), lambda j: (0, j)),
            pl.BlockSpec((D1, C), lambda j: (0, 0)),
            pl.BlockSpec((D1, 1), lambda j: (0, 0)),
            pl.BlockSpec((D1, D1), lambda j: (0, 0)),
            pl.BlockSpec((D1, 1), lambda j: (0, 0)),
            pl.BlockSpec((1, D1), lambda j: (0, 0)),
            pl.BlockSpec((P, C), lambda j: (0, 0)),
        ],
        out_specs=pl.BlockSpec((P, C), lambda j: (0, 0)),
        out_shape=jax.ShapeDtypeStruct((P, C), jnp.float32),
        scratch_shapes=[
            pltpu.VMEM((P, 128), jnp.float32),
            pltpu.VMEM((P, 128), jnp.float32),
            pltpu.VMEM((P, C), jnp.float32),
            pltpu.VMEM((P, 128), jnp.float32),
        ],
    )(Y2, A2, W1, b1.reshape(D1, 1), W2, b2.reshape(D1, 1),
      W3, M.T)

    yhat2 = pl.pallas_call(
        _recon_kernel,
        grid=(nb2,),
        in_specs=[
            pl.BlockSpec((P, C), lambda j: (0, 0)),
            pl.BlockSpec((P, nblk2), lambda j: (0, j)),
        ],
        out_specs=pl.BlockSpec((C, nblk2), lambda j: (0, j)),
        out_shape=jax.ShapeDtypeStruct((C, N), jnp.float32),
    )(mc_t, A2)

    return yhat2.reshape(B, C, H, W_), mc_t.T


# speedup vs baseline: 5.1290x; 2.1039x over previous
import functools

import jax
import jax.numpy as jnp
from jax.experimental import pallas as pl
from jax.experimental.pallas import tpu as pltpu

_NEG = -1e30


def _stats_kernel(y_ref, a_ref, w1_ref, b1_ref, w2_ref, b2_ref, w3_ref,
                  mt_ref, mc_ref, m_s, s_s, v_s, c_s, *, nb, pcls):
    j = pl.program_id(0)

    @pl.when(j == 0)
    def _init():
        m_s[...] = jnp.full(m_s.shape, _NEG, jnp.float32)
        s_s[...] = jnp.zeros(s_s.shape, jnp.float32)
        v_s[...] = jnp.zeros(v_s.shape, jnp.float32)
        c_s[...] = jnp.zeros(c_s.shape, jnp.float32)

    cdim, bh, wdim = y_ref.shape
    npix = bh * wdim
    y = y_ref[...].reshape(cdim, npix)
    a = a_ref[...].reshape(pcls, npix)

    h = jnp.maximum(
        jnp.dot(w1_ref[...], y, preferred_element_type=jnp.float32)
        + b1_ref[...], 0.0)
    h = jnp.maximum(
        jnp.dot(w2_ref[...], h, preferred_element_type=jnp.float32)
        + b2_ref[...], 0.0)
    logits = jnp.dot(w3_ref[...], h, preferred_element_type=jnp.float32)

    amax = jnp.max(a, axis=0, keepdims=True)
    iota = jax.lax.broadcasted_iota(jnp.int32, a.shape, 0)
    idx = jnp.min(jnp.where(a == amax, iota, pcls), axis=0, keepdims=True)
    onehot = iota == idx

    lmask = jnp.where(onehot, logits, _NEG)
    bm = jnp.max(lmask, axis=1, keepdims=True)
    m_old = m_s[:, 0:1]
    m_new = jnp.maximum(m_old, bm)
    alpha = jnp.exp(m_old - m_new)
    p = jnp.exp(jnp.where(onehot, logits - m_new, _NEG))
    s_new = s_s[:, 0:1] * alpha + jnp.sum(p, axis=1, keepdims=True)
    v_new = v_s[...] * alpha + jax.lax.dot_general(
        p, y, (((1,), (1,)), ((), ())), preferred_element_type=jnp.float32)
    c_new = c_s[:, 0:1] + jnp.sum(onehot.astype(jnp.float32), axis=1,
                                  keepdims=True)

    m_s[...] = jnp.broadcast_to(m_new, m_s.shape)
    s_s[...] = jnp.broadcast_to(s_new, s_s.shape)
    v_s[...] = v_new
    c_s[...] = jnp.broadcast_to(c_new, c_s.shape)

    @pl.when(j == nb - 1)
    def _fin():
        w = v_s[...] / s_s[:, 0:1]
        mb = jnp.maximum(mt_ref[...], 0.0)
        col = jnp.where(c_s[:, 0:1] > 10.0, 0.5 * w + 0.5 * mb, mb)
        mc_ref[...] = jnp.clip(col, 0.0, 2.0)


def _recon_kernel(mc_ref, a_ref, out_ref):
    cdim, bh, wdim = out_ref.shape
    pcls = a_ref.shape[0]
    a = a_ref[...].reshape(pcls, bh * wdim)
    res = jax.lax.dot_general(
        mc_ref[...], a, (((0,), (0,)), ((), ())),
        preferred_element_type=jnp.float32)
    out_ref[...] = res.reshape(cdim, bh, wdim)


@jax.jit
def kernel(abundances, Y, M, W1, b1, W2, b2, W3, b3):
    B, P, H, W_ = abundances.shape
    C = Y.shape[1]
    A3 = abundances.reshape(P, H, W_)
    Y3 = Y.reshape(C, H, W_)
    D1 = W1.shape[0]

    bh1 = 16
    nb = H // bh1
    bh2 = 32
    nb2 = H // bh2

    mc_t = pl.pallas_call(
        functools.partial(_stats_kernel, nb=nb, pcls=P),
        grid=(nb,),
        in_specs=[
            pl.BlockSpec((C, bh1, W_), lambda j: (0, j, 0)),
            pl.BlockSpec((P, bh1, W_), lambda j: (0, j, 0)),
            pl.BlockSpec((D1, C), lambda j: (0, 0)),
            pl.BlockSpec((D1, 1), lambda j: (0, 0)),
            pl.BlockSpec((D1, D1), lambda j: (0, 0)),
            pl.BlockSpec((D1, 1), lambda j: (0, 0)),
            pl.BlockSpec((1, D1), lambda j: (0, 0)),
            pl.BlockSpec((P, C), lambda j: (0, 0)),
        ],
        out_specs=pl.BlockSpec((P, C), lambda j: (0, 0)),
        out_shape=jax.ShapeDtypeStruct((P, C), jnp.float32),
        scratch_shapes=[
            pltpu.VMEM((P, 128), jnp.float32),
            pltpu.VMEM((P, 128), jnp.float32),
            pltpu.VMEM((P, C), jnp.float32),
            pltpu.VMEM((P, 128), jnp.float32),
        ],
    )(Y3, A3, W1, b1.reshape(D1, 1), W2, b2.reshape(D1, 1),
      W3, M.T)

    yhat3 = pl.pallas_call(
        _recon_kernel,
        grid=(nb2,),
        in_specs=[
            pl.BlockSpec((P, C), lambda j: (0, 0)),
            pl.BlockSpec((P, bh2, W_), lambda j: (0, j, 0)),
        ],
        out_specs=pl.BlockSpec((C, bh2, W_), lambda j: (0, j, 0)),
        out_shape=jax.ShapeDtypeStruct((C, H, W_), jnp.float32),
    )(mc_t, A3)

    return yhat3.reshape(B, C, H, W_), mc_t.T


# speedup vs baseline: 5.6624x; 1.1040x over previous
import functools

import jax
import jax.numpy as jnp
from jax.experimental import pallas as pl
from jax.experimental.pallas import tpu as pltpu

_NEG = -1e30


def _stats_kernel(y_ref, a_ref, w1_ref, b1_ref, w2_ref, b2_ref, w3_ref,
                  mt_ref, mc_ref, m_s, s_s, v_s, c_s, *, nb, pcls):
    j = pl.program_id(0)

    @pl.when(j == 0)
    def _init():
        m_s[...] = jnp.full(m_s.shape, _NEG, jnp.float32)
        s_s[...] = jnp.zeros(s_s.shape, jnp.float32)
        v_s[...] = jnp.zeros(v_s.shape, jnp.float32)
        c_s[...] = jnp.zeros(c_s.shape, jnp.float32)

    cdim, bh, wdim = y_ref.shape
    npix = bh * wdim
    y = y_ref[...].astype(jnp.bfloat16).reshape(cdim, npix)
    a = a_ref[...].reshape(pcls, npix)

    h = jnp.maximum(
        jnp.dot(w1_ref[...].astype(jnp.bfloat16), y,
                preferred_element_type=jnp.float32)
        + b1_ref[...], 0.0)
    h = jnp.maximum(
        jnp.dot(w2_ref[...].astype(jnp.bfloat16), h.astype(jnp.bfloat16),
                preferred_element_type=jnp.float32)
        + b2_ref[...], 0.0)
    logits = jnp.dot(w3_ref[...].astype(jnp.bfloat16),
                     h.astype(jnp.bfloat16),
                     preferred_element_type=jnp.float32)

    amax = jnp.max(a, axis=0, keepdims=True)
    iota = jax.lax.broadcasted_iota(jnp.int32, a.shape, 0)
    idx = jnp.min(jnp.where(a == amax, iota, pcls), axis=0, keepdims=True)
    onehot = iota == idx

    lmask = jnp.where(onehot, logits, _NEG)
    bm = jnp.max(lmask, axis=1, keepdims=True)
    m_old = m_s[:, 0:1]
    m_new = jnp.maximum(m_old, bm)
    alpha = jnp.exp(m_old - m_new)
    p = jnp.exp(jnp.where(onehot, logits - m_new, _NEG))
    s_new = s_s[:, 0:1] * alpha + jnp.sum(p, axis=1, keepdims=True)
    v_new = v_s[...] * alpha + jax.lax.dot_general(
        p.astype(jnp.bfloat16), y, (((1,), (1,)), ((), ())),
        preferred_element_type=jnp.float32)
    c_new = c_s[:, 0:1] + jnp.sum(onehot.astype(jnp.float32), axis=1,
                                  keepdims=True)

    m_s[...] = jnp.broadcast_to(m_new, m_s.shape)
    s_s[...] = jnp.broadcast_to(s_new, s_s.shape)
    v_s[...] = v_new
    c_s[...] = jnp.broadcast_to(c_new, c_s.shape)

    @pl.when(j == nb - 1)
    def _fin():
        w = v_s[...] / s_s[:, 0:1]
        mb = jnp.maximum(mt_ref[...], 0.0)
        col = jnp.where(c_s[:, 0:1] > 10.0, 0.5 * w + 0.5 * mb, mb)
        mc_ref[...] = jnp.clip(col, 0.0, 2.0)


def _recon_kernel(mc_ref, a_ref, out_ref):
    cdim, bh, wdim = out_ref.shape
    pcls = a_ref.shape[0]
    a = a_ref[...].reshape(pcls, bh * wdim)
    res = jax.lax.dot_general(
        mc_ref[...], a, (((0,), (0,)), ((), ())),
        preferred_element_type=jnp.float32)
    out_ref[...] = res.reshape(cdim, bh, wdim)


@jax.jit
def kernel(abundances, Y, M, W1, b1, W2, b2, W3, b3):
    B, P, H, W_ = abundances.shape
    C = Y.shape[1]
    A3 = abundances.reshape(P, H, W_)
    Y3 = Y.reshape(C, H, W_)
    D1 = W1.shape[0]

    bh1 = 16
    nb = H // bh1
    bh2 = 32
    nb2 = H // bh2

    mc_t = pl.pallas_call(
        functools.partial(_stats_kernel, nb=nb, pcls=P),
        grid=(nb,),
        in_specs=[
            pl.BlockSpec((C, bh1, W_), lambda j: (0, j, 0)),
            pl.BlockSpec((P, bh1, W_), lambda j: (0, j, 0)),
            pl.BlockSpec((D1, C), lambda j: (0, 0)),
            pl.BlockSpec((D1, 1), lambda j: (0, 0)),
            pl.BlockSpec((D1, D1), lambda j: (0, 0)),
            pl.BlockSpec((D1, 1), lambda j: (0, 0)),
            pl.BlockSpec((1, D1), lambda j: (0, 0)),
            pl.BlockSpec((P, C), lambda j: (0, 0)),
        ],
        out_specs=pl.BlockSpec((P, C), lambda j: (0, 0)),
        out_shape=jax.ShapeDtypeStruct((P, C), jnp.float32),
        scratch_shapes=[
            pltpu.VMEM((P, 128), jnp.float32),
            pltpu.VMEM((P, 128), jnp.float32),
            pltpu.VMEM((P, C), jnp.float32),
            pltpu.VMEM((P, 128), jnp.float32),
        ],
    )(Y3, A3, W1, b1.reshape(D1, 1), W2, b2.reshape(D1, 1),
      W3, M.T)

    yhat3 = pl.pallas_call(
        _recon_kernel,
        grid=(nb2,),
        in_specs=[
            pl.BlockSpec((P, C), lambda j: (0, 0)),
            pl.BlockSpec((P, bh2, W_), lambda j: (0, j, 0)),
        ],
        out_specs=pl.BlockSpec((C, bh2, W_), lambda j: (0, j, 0)),
        out_shape=jax.ShapeDtypeStruct((C, H, W_), jnp.float32),
    )(mc_t, A3)

    return yhat3.reshape(B, C, H, W_), mc_t.T


# speedup vs baseline: 5.7222x; 1.0106x over previous
import functools

import jax
import jax.numpy as jnp
from jax.experimental import pallas as pl
from jax.experimental.pallas import tpu as pltpu

_NEG = -1e30


def _fused_kernel(y_ref, a_ref, w1_ref, b1_ref, w2_ref, b2_ref, w3_ref,
                  mt_ref, mc_ref, yhat_ref,
                  m_s, s_s, v_s, c_s, af_s, mcs_s, *, nb, pcls):
    j = pl.program_id(0)

    @pl.when(j == 0)
    def _init():
        m_s[...] = jnp.full(m_s.shape, _NEG, jnp.float32)
        s_s[...] = jnp.zeros(s_s.shape, jnp.float32)
        v_s[...] = jnp.zeros(v_s.shape, jnp.float32)
        c_s[...] = jnp.zeros(c_s.shape, jnp.float32)

    cdim, bh, wdim = y_ref.shape
    npix = bh * wdim

    @pl.when(j < nb)
    def _stats():
        y = y_ref[...].astype(jnp.bfloat16).reshape(cdim, npix)
        a = a_ref[...].reshape(pcls, npix)
        af_s[:, pl.ds(j * npix, npix)] = a

        h = jnp.maximum(
            jnp.dot(w1_ref[...].astype(jnp.bfloat16), y,
                    preferred_element_type=jnp.float32)
            + b1_ref[...], 0.0)
        h = jnp.maximum(
            jnp.dot(w2_ref[...].astype(jnp.bfloat16), h.astype(jnp.bfloat16),
                    preferred_element_type=jnp.float32)
            + b2_ref[...], 0.0)
        logits = jnp.dot(w3_ref[...].astype(jnp.bfloat16),
                         h.astype(jnp.bfloat16),
                         preferred_element_type=jnp.float32)

        amax = jnp.max(a, axis=0, keepdims=True)
        iota = jax.lax.broadcasted_iota(jnp.int32, a.shape, 0)
        idx = jnp.min(jnp.where(a == amax, iota, pcls), axis=0, keepdims=True)
        onehot = iota == idx

        lmask = jnp.where(onehot, logits, _NEG)
        bm = jnp.max(lmask, axis=1, keepdims=True)
        m_old = m_s[:, 0:1]
        m_new = jnp.maximum(m_old, bm)
        alpha = jnp.exp(m_old - m_new)
        p = jnp.exp(jnp.where(onehot, logits - m_new, _NEG))
        s_new = s_s[:, 0:1] * alpha + jnp.sum(p, axis=1, keepdims=True)
        v_new = v_s[...] * alpha + jax.lax.dot_general(
            p.astype(jnp.bfloat16), y, (((1,), (1,)), ((), ())),
            preferred_element_type=jnp.float32)
        c_new = c_s[:, 0:1] + jnp.sum(onehot.astype(jnp.float32), axis=1,
                                      keepdims=True)

        m_s[...] = jnp.broadcast_to(m_new, m_s.shape)
        s_s[...] = jnp.broadcast_to(s_new, s_s.shape)
        v_s[...] = v_new
        c_s[...] = jnp.broadcast_to(c_new, c_s.shape)

        @pl.when(j == nb - 1)
        def _fin():
            w = v_s[...] / s_s[:, 0:1]
            mb = jnp.maximum(mt_ref[...], 0.0)
            col = jnp.where(c_s[:, 0:1] > 10.0, 0.5 * w + 0.5 * mb, mb)
            mc = jnp.clip(col, 0.0, 2.0)
            mcs_s[...] = mc
            mc_ref[...] = mc

    @pl.when(j >= nb)
    def _recon():
        k = j - nb
        a = af_s[:, pl.ds(k * npix, npix)]
        res = jax.lax.dot_general(
            mcs_s[...], a, (((0,), (0,)), ((), ())),
            preferred_element_type=jnp.float32)
        yhat_ref[...] = res.reshape(cdim, bh, wdim)


@jax.jit
def kernel(abundances, Y, M, W1, b1, W2, b2, W3, b3):
    B, P, H, W_ = abundances.shape
    C = Y.shape[1]
    A3 = abundances.reshape(P, H, W_)
    Y3 = Y.reshape(C, H, W_)
    D1 = W1.shape[0]

    bh = 16
    nb = H // bh
    npix = bh * W_

    mc_t, yhat3 = pl.pallas_call(
        functools.partial(_fused_kernel, nb=nb, pcls=P),
        grid=(2 * nb,),
        in_specs=[
            pl.BlockSpec((C, bh, W_), lambda j: (0, jnp.minimum(j, nb - 1), 0)),
            pl.BlockSpec((P, bh, W_), lambda j: (0, jnp.minimum(j, nb - 1), 0)),
            pl.BlockSpec((D1, C), lambda j: (0, 0)),
            pl.BlockSpec((D1, 1), lambda j: (0, 0)),
            pl.BlockSpec((D1, D1), lambda j: (0, 0)),
            pl.BlockSpec((D1, 1), lambda j: (0, 0)),
            pl.BlockSpec((1, D1), lambda j: (0, 0)),
            pl.BlockSpec((P, C), lambda j: (0, 0)),
        ],
        out_specs=[
            pl.BlockSpec((P, C), lambda j: (0, 0)),
            pl.BlockSpec((C, bh, W_), lambda j: (0, jnp.maximum(j - nb, 0), 0)),
        ],
        out_shape=[
            jax.ShapeDtypeStruct((P, C), jnp.float32),
            jax.ShapeDtypeStruct((C, H, W_), jnp.float32),
        ],
        scratch_shapes=[
            pltpu.VMEM((P, 128), jnp.float32),
            pltpu.VMEM((P, 128), jnp.float32),
            pltpu.VMEM((P, C), jnp.float32),
            pltpu.VMEM((P, 128), jnp.float32),
            pltpu.VMEM((P, H * W_), jnp.float32),
            pltpu.VMEM((P, C), jnp.float32),
        ],
    )(Y3, A3, W1, b1.reshape(D1, 1), W2, b2.reshape(D1, 1), W3, M.T)

    return yhat3.reshape(B, C, H, W_), mc_t.T
